# baseline (speedup 1.0000x reference)
import functools

import jax
import jax.numpy as jnp
from jax import lax
from jax.experimental import pallas as pl
from jax.experimental.pallas import tpu as pltpu
from jax.experimental.pallas import tpu_sc as plsc

E = 8
D = 1024
H = 1024
N = 8192

BLK = 256
NB = N // BLK + E
NPAD = NB * BLK

NC = 2
NS = 16
NW = NC * NS
GCH = 16
NCH = N // NW // GCH
NBUF = 4


def _routing(ids):
    oh = (ids[:, None] == jnp.arange(E, dtype=jnp.int32)[None, :])
    csum = jnp.cumsum(oh.astype(jnp.int32), axis=0)
    counts = csum[-1]
    bcounts = (counts + BLK - 1) // BLK
    block_ends = jnp.cumsum(bcounts)
    padded_starts = (block_ends - bcounts) * BLK
    pos = jnp.sum(jnp.where(oh, csum - 1 + padded_starts[None, :], 0),
                  axis=1).astype(jnp.int32)
    blk_ids = jnp.arange(NB, dtype=jnp.int32)
    block_expert = jnp.sum(
        (blk_ids[:, None] >= block_ends[None, :]).astype(jnp.int32), axis=1)
    block_expert = jnp.minimum(block_expert, E - 1).astype(jnp.int32)

    is_new = jnp.concatenate(
        [jnp.ones((1,), jnp.int32),
         (block_expert[1:] != block_expert[:-1]).astype(jnp.int32)])
    ordn = jnp.cumsum(is_new) - 1
    slot_of = (ordn % 3).astype(jnp.int32)
    big = jnp.int32(1 << 20)
    key = jnp.where(is_new == 1, blk_ids * 16 + block_expert, big)
    suff_inc = jnp.flip(lax.cummin(jnp.flip(key)))
    suff_exc = jnp.concatenate([suff_inc[1:], big[None]])
    valid1 = (is_new == 1) & (suff_exc < big)
    nxt_blk = jnp.minimum(suff_exc // 16, NB - 1)
    suff2 = suff_exc[nxt_blk]
    valid2 = valid1 & (suff2 < big)
    fetch_flag = valid2.astype(jnp.int32)
    fetch_e = (suff2 % 16).astype(jnp.int32)
    fetch_slot = ((slot_of + 2) % 3).astype(jnp.int32)
    f1_flag = valid1.astype(jnp.int32)
    f1_e = (suff_exc % 16).astype(jnp.int32)
    f1_slot = ((slot_of + 1) % 3).astype(jnp.int32)
    sched = (block_expert, slot_of, is_new, fetch_flag, fetch_e, fetch_slot,
             f1_flag, f1_e, f1_slot)
    return pos, sched


@functools.cache
def _sc_kernels():
    mesh = plsc.VectorSubcoreMesh(
        core_axis_name="c", subcore_axis_name="s",
        num_cores=NC, num_subcores=NS)

    sems = [pltpu.SemaphoreType.DMA] * (2 * NBUF)

    @functools.partial(
        pl.kernel,
        out_type=jax.ShapeDtypeStruct((NPAD, D), jnp.float32),
        mesh=mesh,
        scratch_types=[
            pltpu.VMEM((NCH, GCH), jnp.int32),
            pltpu.VMEM((NBUF, GCH, D), jnp.float32),
            *sems,
        ],
    )
    def sc_dispatch(pos_hbm, x_hbm, out_hbm, idx_v, rows_v, *sem):
        wid = lax.axis_index("s") * NC + lax.axis_index("c")
        base = wid * (N // NW)
        pltpu.sync_copy(pos_hbm.at[pl.ds(wid * NCH, NCH)], idx_v)

        loads = [None] * NBUF
        scats = [None] * NBUF

        def start_load(c):
            b = c % NBUF
            loads[b] = pltpu.async_copy(
                x_hbm.at[pl.ds(base + c * GCH, GCH)], rows_v.at[b], sem[b])

        start_load(0)
        start_load(1)
        for c in range(NCH):
            b = c % NBUF
            loads[b].wait()
            scats[b] = pltpu.async_copy(
                rows_v.at[b], out_hbm.at[idx_v.at[c]], sem[NBUF + b])
            nxt = c + 2
            if nxt < NCH:
                b2 = nxt % NBUF
                if scats[b2] is not None:
                    scats[b2].wait()
                start_load(nxt)
        for b in range(NBUF):
            scats[b].wait()

    @functools.partial(
        pl.kernel,
        out_type=(
            jax.ShapeDtypeStruct((N, H), jnp.float32),
            jax.ShapeDtypeStruct((N // GCH, GCH), jnp.float32),
        ),
        mesh=mesh,
        scratch_types=[
            pltpu.VMEM((NCH, GCH), jnp.int32),
            pltpu.VMEM((NBUF, GCH, H), jnp.float32),
            pltpu.VMEM((NCH, GCH), jnp.float32),
            pltpu.SemaphoreType.DMA,
            *sems,
        ],
    )
    def sc_combine(pos_hbm, act_hbm, val_hbm, out_act_hbm, out_val_hbm,
                   idx_v, rows_v, vals_v, sem_v, *sem):
        wid = lax.axis_index("s") * NC + lax.axis_index("c")
        base = wid * (N // NW)
        pltpu.sync_copy(pos_hbm.at[pl.ds(wid * NCH, NCH)], idx_v)

        vcps = [pltpu.async_copy(val_hbm.at[idx_v.at[c]], vals_v.at[c], sem_v)
                for c in range(NCH)]

        gats = [None] * NBUF
        wbs = [None] * NBUF

        def start_gather(c):
            b = c % NBUF
            gats[b] = pltpu.async_copy(
                act_hbm.at[idx_v.at[c]], rows_v.at[b], sem[b])

        start_gather(0)
        start_gather(1)
        for c in range(NCH):
            b = c % NBUF
            gats[b].wait()
            wbs[b] = pltpu.async_copy(
                rows_v.at[b], out_act_hbm.at[pl.ds(base + c * GCH, GCH)],
                sem[NBUF + b])
            nxt = c + 2
            if nxt < NCH:
                b2 = nxt % NBUF
                if wbs[b2] is not None:
                    wbs[b2].wait()
                start_gather(nxt)
        for cp in vcps:
            cp.wait()
        pltpu.sync_copy(vals_v, out_val_hbm.at[pl.ds(wid * NCH, NCH)])
        for b in range(NBUF):
            wbs[b].wait()

    return sc_dispatch, sc_combine


def _tc_body(be_ref, slot_ref, isnew_ref, fflag_ref, fe_ref, fslot_ref,
             f1flag_ref, f1e_ref, f1slot_ref, vb_ref,
             x_ref, aW1_ref, ab1_ref, aW2_ref, ab2_ref,
             cW1_ref, cb1_ref, cW2_ref, cb2_ref, vW_ref, act_ref, val_ref,
             r_a1, r_a2, r_c1, r_c2, sem):
    i = pl.program_id(0)
    e = be_ref[i]
    slot = slot_ref[i]
    whbms = (aW1_ref, aW2_ref, cW1_ref, cW2_ref)
    rings = (r_a1, r_a2, r_c1, r_c2)

    def wcopy(whbm, ring, src_e, dst_slot):
        return pltpu.make_async_copy(
            whbm.at[src_e], ring.at[dst_slot], sem.at[dst_slot])

    @pl.when(i == 0)
    def _fetch_first():
        for whbm, ring in zip(whbms, rings):
            wcopy(whbm, ring, e, slot).start()

    @pl.when(jnp.logical_and(i == 0, f1flag_ref[0] == 1))
    def _prime_second():
        for whbm, ring in zip(whbms, rings):
            wcopy(whbm, ring, f1e_ref[0], f1slot_ref[0]).start()

    @pl.when(fflag_ref[i] == 1)
    def _fetch_second_next_expert():
        fe = fe_ref[i]
        fs = fslot_ref[i]
        for whbm, ring in zip(whbms, rings):
            wcopy(whbm, ring, fe, fs).start()

    @pl.when(isnew_ref[i] == 1)
    def _await_weights():
        for whbm, ring in zip(whbms, rings):
            wcopy(whbm, ring, e, slot).wait()

    x = x_ref[...]
    ha = jnp.tanh(jnp.dot(x, r_a1[slot], preferred_element_type=jnp.float32)
                  + ab1_ref[0])
    act = jnp.tanh(jnp.dot(ha, r_a2[slot], preferred_element_type=jnp.float32)
                   + ab2_ref[0])
    hc = jnp.tanh(jnp.dot(x, r_c1[slot], preferred_element_type=jnp.float32)
                  + cb1_ref[0])
    c2 = jnp.tanh(jnp.dot(hc, r_c2[slot], preferred_element_type=jnp.float32)
                  + cb2_ref[0])
    val = jnp.sum(c2 * vW_ref[0], axis=1, keepdims=True) + vb_ref[e]
    act_ref[...] = act
    val_ref[...] = val


def _tc_experts(sched, vb_flat, x_pad, aW1, ab1, aW2, ab2,
                cW1, cb1, cW2, cb2, vW):
    nsp = len(sched) + 1
    sp = ["_"] * nsp

    def bias_spec():
        return pl.BlockSpec((1, 1, H), lambda i, *sp: (sp[0][i], 0, 0))

    grid_spec = pltpu.PrefetchScalarGridSpec(
        num_scalar_prefetch=nsp,
        grid=(NB,),
        in_specs=[
            pl.BlockSpec((BLK, D), lambda i, *sp: (i, 0)),
            pl.BlockSpec(memory_space=pl.ANY),
            bias_spec(),
            pl.BlockSpec(memory_space=pl.ANY),
            bias_spec(),
            pl.BlockSpec(memory_space=pl.ANY),
            bias_spec(),
            pl.BlockSpec(memory_space=pl.ANY),
            bias_spec(),
            bias_spec(),
        ],
        out_specs=[
            pl.BlockSpec((BLK, H), lambda i, *sp: (i, 0)),
            pl.BlockSpec((BLK, 1), lambda i, *sp: (i, 0)),
        ],
        scratch_shapes=[
            pltpu.VMEM((3, D, H), jnp.float32),
            pltpu.VMEM((3, H, H), jnp.float32),
            pltpu.VMEM((3, D, H), jnp.float32),
            pltpu.VMEM((3, H, H), jnp.float32),
            pltpu.SemaphoreType.DMA((3,)),
        ],
    )
    return pl.pallas_call(
        _tc_body,
        grid_spec=grid_spec,
        out_shape=[
            jax.ShapeDtypeStruct((NPAD, H), jnp.float32),
            jax.ShapeDtypeStruct((NPAD, 1), jnp.float32),
        ],
        compiler_params=pltpu.CompilerParams(
            dimension_semantics=("arbitrary",),
        ),
    )(*sched, vb_flat,
      x_pad,
      aW1, ab1.reshape(E, 1, H),
      aW2, ab2.reshape(E, 1, H),
      cW1, cb1.reshape(E, 1, H),
      cW2, cb2.reshape(E, 1, H),
      vW.reshape(E, 1, H))


def kernel(controller_ids, inputs, rnn_hxs, masks, aW1, ab1, aW2, ab2,
           cW1, cb1, cW2, cb2, vW, vb):
    ids = controller_ids.astype(jnp.int32)
    pos, sched = _routing(ids)

    sc_dispatch, sc_combine = _sc_kernels()
    pos2d = pos.reshape(N // GCH, GCH)
    x_pad = sc_dispatch(pos2d, inputs)
    act_pad, val_pad = _tc_experts(
        sched, vb.reshape(E), x_pad,
        aW1, ab1, aW2, ab2, cW1, cb1, cW2, cb2, vW)
    actor, value = sc_combine(pos2d, act_pad, val_pad.reshape(NPAD))

    return (value.reshape(N, 1), actor, rnn_hxs)


# speedup vs baseline: 1.0173x; 1.0173x over previous
import functools

import jax
import jax.numpy as jnp
from jax import lax
from jax.experimental import pallas as pl
from jax.experimental.pallas import tpu as pltpu
from jax.experimental.pallas import tpu_sc as plsc

E = 8
D = 1024
H = 1024
N = 8192

BLK = 256
NB = N // BLK + E
NPAD = NB * BLK

NC = 2
NS = 16
NW = NC * NS
GCH = 16
NCH = N // NW // GCH
NBUF = 4


def _routing(ids):
    oh = (ids[:, None] == jnp.arange(E, dtype=jnp.int32)[None, :])
    csum = jnp.cumsum(oh.astype(jnp.int32), axis=0)
    counts = csum[-1]
    bcounts = (counts + BLK - 1) // BLK
    block_ends = jnp.cumsum(bcounts)
    padded_starts = (block_ends - bcounts) * BLK
    pos = jnp.sum(jnp.where(oh, csum - 1 + padded_starts[None, :], 0),
                  axis=1).astype(jnp.int32)
    blk_ids = jnp.arange(NB, dtype=jnp.int32)
    block_expert = jnp.sum(
        (blk_ids[:, None] >= block_ends[None, :]).astype(jnp.int32), axis=1)
    block_expert = jnp.minimum(block_expert, E - 1).astype(jnp.int32)

    is_new = jnp.concatenate(
        [jnp.ones((1,), jnp.int32),
         (block_expert[1:] != block_expert[:-1]).astype(jnp.int32)])
    ordn = jnp.cumsum(is_new) - 1
    slot_of = (ordn % 2).astype(jnp.int32)
    big = jnp.int32(1 << 20)
    key = jnp.where(is_new == 1, blk_ids * 16 + block_expert, big)
    suff_inc = jnp.flip(lax.cummin(jnp.flip(key)))
    suff_exc = jnp.concatenate([suff_inc[1:], big[None]])
    valid1 = (is_new == 1) & (suff_exc < big)
    fetch_flag = valid1.astype(jnp.int32)
    fetch_e = (suff_exc % 16).astype(jnp.int32)
    fetch_slot = ((slot_of + 1) % 2).astype(jnp.int32)
    f1_flag = jnp.zeros((NB,), jnp.int32)
    f1_e = fetch_e
    f1_slot = fetch_slot
    sched = (block_expert, slot_of, is_new, fetch_flag, fetch_e, fetch_slot,
             f1_flag, f1_e, f1_slot)
    return pos, sched


@functools.cache
def _sc_kernels():
    mesh = plsc.VectorSubcoreMesh(
        core_axis_name="c", subcore_axis_name="s",
        num_cores=NC, num_subcores=NS)

    sems = [pltpu.SemaphoreType.DMA] * (2 * NBUF)

    @functools.partial(
        pl.kernel,
        out_type=jax.ShapeDtypeStruct((NPAD, D), jnp.float32),
        mesh=mesh,
        scratch_types=[
            pltpu.VMEM((NCH, GCH), jnp.int32),
            pltpu.VMEM((NBUF, GCH, D), jnp.float32),
            *sems,
        ],
    )
    def sc_dispatch(pos_hbm, x_hbm, out_hbm, idx_v, rows_v, *sem):
        wid = lax.axis_index("s") * NC + lax.axis_index("c")
        base = wid * (N // NW)
        pltpu.sync_copy(pos_hbm.at[pl.ds(wid * NCH, NCH)], idx_v)

        loads = [None] * NBUF
        scats = [None] * NBUF

        def start_load(c):
            b = c % NBUF
            loads[b] = pltpu.async_copy(
                x_hbm.at[pl.ds(base + c * GCH, GCH)], rows_v.at[b], sem[b])

        start_load(0)
        start_load(1)
        for c in range(NCH):
            b = c % NBUF
            loads[b].wait()
            scats[b] = pltpu.async_copy(
                rows_v.at[b], out_hbm.at[idx_v.at[c]], sem[NBUF + b])
            nxt = c + 2
            if nxt < NCH:
                b2 = nxt % NBUF
                if scats[b2] is not None:
                    scats[b2].wait()
                start_load(nxt)
        for b in range(NBUF):
            scats[b].wait()

    @functools.partial(
        pl.kernel,
        out_type=(
            jax.ShapeDtypeStruct((N, H), jnp.float32),
            jax.ShapeDtypeStruct((N // GCH, GCH), jnp.float32),
        ),
        mesh=mesh,
        scratch_types=[
            pltpu.VMEM((NCH, GCH), jnp.int32),
            pltpu.VMEM((NBUF, GCH, H), jnp.float32),
            pltpu.VMEM((NCH, GCH), jnp.float32),
            pltpu.SemaphoreType.DMA,
            *sems,
        ],
    )
    def sc_combine(pos_hbm, act_hbm, val_hbm, out_act_hbm, out_val_hbm,
                   idx_v, rows_v, vals_v, sem_v, *sem):
        wid = lax.axis_index("s") * NC + lax.axis_index("c")
        base = wid * (N // NW)
        pltpu.sync_copy(pos_hbm.at[pl.ds(wid * NCH, NCH)], idx_v)

        vcps = [pltpu.async_copy(val_hbm.at[idx_v.at[c]], vals_v.at[c], sem_v)
                for c in range(NCH)]

        gats = [None] * NBUF
        wbs = [None] * NBUF

        def start_gather(c):
            b = c % NBUF
            gats[b] = pltpu.async_copy(
                act_hbm.at[idx_v.at[c]], rows_v.at[b], sem[b])

        start_gather(0)
        start_gather(1)
        for c in range(NCH):
            b = c % NBUF
            gats[b].wait()
            wbs[b] = pltpu.async_copy(
                rows_v.at[b], out_act_hbm.at[pl.ds(base + c * GCH, GCH)],
                sem[NBUF + b])
            nxt = c + 2
            if nxt < NCH:
                b2 = nxt % NBUF
                if wbs[b2] is not None:
                    wbs[b2].wait()
                start_gather(nxt)
        for cp in vcps:
            cp.wait()
        pltpu.sync_copy(vals_v, out_val_hbm.at[pl.ds(wid * NCH, NCH)])
        for b in range(NBUF):
            wbs[b].wait()

    return sc_dispatch, sc_combine


def _tc_body(be_ref, slot_ref, isnew_ref, fflag_ref, fe_ref, fslot_ref,
             f1flag_ref, f1e_ref, f1slot_ref, vb_ref,
             x_ref, aW1_ref, ab1_ref, aW2_ref, ab2_ref,
             cW1_ref, cb1_ref, cW2_ref, cb2_ref, vW_ref, act_ref, val_ref,
             r_a1, r_a2, r_c1, r_c2, sem):
    i = pl.program_id(0)
    e = be_ref[i]
    slot = slot_ref[i]
    whbms = (aW1_ref, aW2_ref, cW1_ref, cW2_ref)
    rings = (r_a1, r_a2, r_c1, r_c2)

    def wcopy(whbm, ring, src_e, dst_slot):
        return pltpu.make_async_copy(
            whbm.at[src_e], ring.at[dst_slot], sem.at[dst_slot])

    @pl.when(i == 0)
    def _fetch_first():
        for whbm, ring in zip(whbms, rings):
            wcopy(whbm, ring, e, slot).start()

    @pl.when(jnp.logical_and(i == 0, f1flag_ref[0] == 1))
    def _prime_second():
        for whbm, ring in zip(whbms, rings):
            wcopy(whbm, ring, f1e_ref[0], f1slot_ref[0]).start()

    @pl.when(fflag_ref[i] == 1)
    def _fetch_second_next_expert():
        fe = fe_ref[i]
        fs = fslot_ref[i]
        for whbm, ring in zip(whbms, rings):
            wcopy(whbm, ring, fe, fs).start()

    @pl.when(isnew_ref[i] == 1)
    def _await_weights():
        for whbm, ring in zip(whbms, rings):
            wcopy(whbm, ring, e, slot).wait()

    x = x_ref[...]
    ha = jnp.tanh(jnp.dot(x, r_a1[slot], preferred_element_type=jnp.float32)
                  + ab1_ref[0])
    act = jnp.tanh(jnp.dot(ha, r_a2[slot], preferred_element_type=jnp.float32)
                   + ab2_ref[0])
    hc = jnp.tanh(jnp.dot(x, r_c1[slot], preferred_element_type=jnp.float32)
                  + cb1_ref[0])
    c2 = jnp.tanh(jnp.dot(hc, r_c2[slot], preferred_element_type=jnp.float32)
                  + cb2_ref[0])
    val = jnp.sum(c2 * vW_ref[0], axis=1, keepdims=True) + vb_ref[e]
    act_ref[...] = act
    val_ref[...] = val


def _tc_experts(sched, vb_flat, x_pad, aW1, ab1, aW2, ab2,
                cW1, cb1, cW2, cb2, vW):
    nsp = len(sched) + 1
    sp = ["_"] * nsp

    def bias_spec():
        return pl.BlockSpec((1, 1, H), lambda i, *sp: (sp[0][i], 0, 0))

    grid_spec = pltpu.PrefetchScalarGridSpec(
        num_scalar_prefetch=nsp,
        grid=(NB,),
        in_specs=[
            pl.BlockSpec((BLK, D), lambda i, *sp: (i, 0)),
            pl.BlockSpec(memory_space=pl.ANY),
            bias_spec(),
            pl.BlockSpec(memory_space=pl.ANY),
            bias_spec(),
            pl.BlockSpec(memory_space=pl.ANY),
            bias_spec(),
            pl.BlockSpec(memory_space=pl.ANY),
            bias_spec(),
            bias_spec(),
        ],
        out_specs=[
            pl.BlockSpec((BLK, H), lambda i, *sp: (i, 0)),
            pl.BlockSpec((BLK, 1), lambda i, *sp: (i, 0)),
        ],
        scratch_shapes=[
            pltpu.VMEM((2, D, H), jnp.float32),
            pltpu.VMEM((2, H, H), jnp.float32),
            pltpu.VMEM((2, D, H), jnp.float32),
            pltpu.VMEM((2, H, H), jnp.float32),
            pltpu.SemaphoreType.DMA((2,)),
        ],
    )
    return pl.pallas_call(
        _tc_body,
        grid_spec=grid_spec,
        out_shape=[
            jax.ShapeDtypeStruct((NPAD, H), jnp.float32),
            jax.ShapeDtypeStruct((NPAD, 1), jnp.float32),
        ],
        compiler_params=pltpu.CompilerParams(
            dimension_semantics=("arbitrary",),
        ),
    )(*sched, vb_flat,
      x_pad,
      aW1, ab1.reshape(E, 1, H),
      aW2, ab2.reshape(E, 1, H),
      cW1, cb1.reshape(E, 1, H),
      cW2, cb2.reshape(E, 1, H),
      vW.reshape(E, 1, H))


def kernel(controller_ids, inputs, rnn_hxs, masks, aW1, ab1, aW2, ab2,
           cW1, cb1, cW2, cb2, vW, vb):
    ids = controller_ids.astype(jnp.int32)
    pos, sched = _routing(ids)

    sc_dispatch, sc_combine = _sc_kernels()
    pos2d = pos.reshape(N // GCH, GCH)
    x_pad = sc_dispatch(pos2d, inputs)
    act_pad, val_pad = _tc_experts(
        sched, vb.reshape(E), x_pad,
        aW1, ab1, aW2, ab2, cW1, cb1, cW2, cb2, vW)
    actor, value = sc_combine(pos2d, act_pad, val_pad.reshape(NPAD))

    return (value.reshape(N, 1), actor, rnn_hxs)


# speedup vs baseline: 1.0224x; 1.0050x over previous
import functools

import jax
import jax.numpy as jnp
from jax import lax
from jax.experimental import pallas as pl
from jax.experimental.pallas import tpu as pltpu
from jax.experimental.pallas import tpu_sc as plsc

E = 8
D = 1024
H = 1024
N = 8192

BLK = 256
NB = N // BLK + E
NPAD = NB * BLK

NC = 2
NS = 16
NW = NC * NS
GCH = 16
NCH = N // NW // GCH
NBUF = 4


def _routing(ids):
    oh = (ids[:, None] == jnp.arange(E, dtype=jnp.int32)[None, :])
    csum = jnp.cumsum(oh.astype(jnp.int16), axis=0,
                      dtype=jnp.int16).astype(jnp.int32)
    counts = csum[-1]
    bcounts = (counts + BLK - 1) // BLK
    block_ends = jnp.cumsum(bcounts)
    padded_starts = (block_ends - bcounts) * BLK
    pos = jnp.sum(jnp.where(oh, csum - 1 + padded_starts[None, :], 0),
                  axis=1).astype(jnp.int32)
    blk_ids = jnp.arange(NB, dtype=jnp.int32)
    block_expert = jnp.sum(
        (blk_ids[:, None] >= block_ends[None, :]).astype(jnp.int32), axis=1)
    block_expert = jnp.minimum(block_expert, E - 1).astype(jnp.int32)

    is_new = jnp.concatenate(
        [jnp.ones((1,), jnp.int32),
         (block_expert[1:] != block_expert[:-1]).astype(jnp.int32)])
    ordn = jnp.cumsum(is_new) - 1
    slot_of = (ordn % 2).astype(jnp.int32)
    big = jnp.int32(1 << 20)
    key = jnp.where(is_new == 1, blk_ids * 16 + block_expert, big)
    suff_inc = jnp.flip(lax.cummin(jnp.flip(key)))
    suff_exc = jnp.concatenate([suff_inc[1:], big[None]])
    valid1 = (is_new == 1) & (suff_exc < big)
    fetch_flag = valid1.astype(jnp.int32)
    fetch_e = (suff_exc % 16).astype(jnp.int32)
    fetch_slot = ((slot_of + 1) % 2).astype(jnp.int32)
    f1_flag = jnp.zeros((NB,), jnp.int32)
    f1_e = fetch_e
    f1_slot = fetch_slot
    sched = (block_expert, slot_of, is_new, fetch_flag, fetch_e, fetch_slot,
             f1_flag, f1_e, f1_slot)
    return pos, sched


@functools.cache
def _sc_kernels():
    mesh = plsc.VectorSubcoreMesh(
        core_axis_name="c", subcore_axis_name="s",
        num_cores=NC, num_subcores=NS)

    sems = [pltpu.SemaphoreType.DMA] * (2 * NBUF)

    @functools.partial(
        pl.kernel,
        out_type=jax.ShapeDtypeStruct((NPAD, D), jnp.float32),
        mesh=mesh,
        scratch_types=[
            pltpu.VMEM((NCH, GCH), jnp.int32),
            pltpu.VMEM((NBUF, GCH, D), jnp.float32),
            *sems,
        ],
    )
    def sc_dispatch(pos_hbm, x_hbm, out_hbm, idx_v, rows_v, *sem):
        wid = lax.axis_index("s") * NC + lax.axis_index("c")
        base = wid * (N // NW)
        pltpu.sync_copy(pos_hbm.at[pl.ds(wid * NCH, NCH)], idx_v)

        loads = [None] * NBUF
        scats = [None] * NBUF

        def start_load(c):
            b = c % NBUF
            loads[b] = pltpu.async_copy(
                x_hbm.at[pl.ds(base + c * GCH, GCH)], rows_v.at[b], sem[b])

        start_load(0)
        start_load(1)
        for c in range(NCH):
            b = c % NBUF
            loads[b].wait()
            scats[b] = pltpu.async_copy(
                rows_v.at[b], out_hbm.at[idx_v.at[c]], sem[NBUF + b])
            nxt = c + 2
            if nxt < NCH:
                b2 = nxt % NBUF
                if scats[b2] is not None:
                    scats[b2].wait()
                start_load(nxt)
        for b in range(NBUF):
            scats[b].wait()

    @functools.partial(
        pl.kernel,
        out_type=(
            jax.ShapeDtypeStruct((N, H), jnp.float32),
            jax.ShapeDtypeStruct((N // GCH, GCH), jnp.float32),
        ),
        mesh=mesh,
        scratch_types=[
            pltpu.VMEM((NCH, GCH), jnp.int32),
            pltpu.VMEM((NBUF, GCH, H), jnp.float32),
            pltpu.VMEM((NCH, GCH), jnp.float32),
            pltpu.SemaphoreType.DMA,
            *sems,
        ],
    )
    def sc_combine(pos_hbm, act_hbm, val_hbm, out_act_hbm, out_val_hbm,
                   idx_v, rows_v, vals_v, sem_v, *sem):
        wid = lax.axis_index("s") * NC + lax.axis_index("c")
        base = wid * (N // NW)
        pltpu.sync_copy(pos_hbm.at[pl.ds(wid * NCH, NCH)], idx_v)

        vcps = [pltpu.async_copy(val_hbm.at[idx_v.at[c]], vals_v.at[c], sem_v)
                for c in range(NCH)]

        gats = [None] * NBUF
        wbs = [None] * NBUF

        def start_gather(c):
            b = c % NBUF
            gats[b] = pltpu.async_copy(
                act_hbm.at[idx_v.at[c]], rows_v.at[b], sem[b])

        start_gather(0)
        start_gather(1)
        for c in range(NCH):
            b = c % NBUF
            gats[b].wait()
            wbs[b] = pltpu.async_copy(
                rows_v.at[b], out_act_hbm.at[pl.ds(base + c * GCH, GCH)],
                sem[NBUF + b])
            nxt = c + 2
            if nxt < NCH:
                b2 = nxt % NBUF
                if wbs[b2] is not None:
                    wbs[b2].wait()
                start_gather(nxt)
        for cp in vcps:
            cp.wait()
        pltpu.sync_copy(vals_v, out_val_hbm.at[pl.ds(wid * NCH, NCH)])
        for b in range(NBUF):
            wbs[b].wait()

    return sc_dispatch, sc_combine


def _tc_body(be_ref, slot_ref, isnew_ref, fflag_ref, fe_ref, fslot_ref,
             f1flag_ref, f1e_ref, f1slot_ref, vb_ref,
             x_ref, aW1_ref, ab1_ref, aW2_ref, ab2_ref,
             cW1_ref, cb1_ref, cW2_ref, cb2_ref, vW_ref, act_ref, val_ref,
             r_a1, r_a2, r_c1, r_c2, sem):
    i = pl.program_id(0)
    e = be_ref[i]
    slot = slot_ref[i]
    whbms = (aW1_ref, aW2_ref, cW1_ref, cW2_ref)
    rings = (r_a1, r_a2, r_c1, r_c2)

    def wcopy(whbm, ring, src_e, dst_slot):
        return pltpu.make_async_copy(
            whbm.at[src_e], ring.at[dst_slot], sem.at[dst_slot])

    @pl.when(i == 0)
    def _fetch_first():
        for whbm, ring in zip(whbms, rings):
            wcopy(whbm, ring, e, slot).start()

    @pl.when(jnp.logical_and(i == 0, f1flag_ref[0] == 1))
    def _prime_second():
        for whbm, ring in zip(whbms, rings):
            wcopy(whbm, ring, f1e_ref[0], f1slot_ref[0]).start()

    @pl.when(fflag_ref[i] == 1)
    def _fetch_second_next_expert():
        fe = fe_ref[i]
        fs = fslot_ref[i]
        for whbm, ring in zip(whbms, rings):
            wcopy(whbm, ring, fe, fs).start()

    @pl.when(isnew_ref[i] == 1)
    def _await_weights():
        for whbm, ring in zip(whbms, rings):
            wcopy(whbm, ring, e, slot).wait()

    x = x_ref[...]
    ha = jnp.tanh(jnp.dot(x, r_a1[slot], preferred_element_type=jnp.float32)
                  + ab1_ref[0])
    act = jnp.tanh(jnp.dot(ha, r_a2[slot], preferred_element_type=jnp.float32)
                   + ab2_ref[0])
    hc = jnp.tanh(jnp.dot(x, r_c1[slot], preferred_element_type=jnp.float32)
                  + cb1_ref[0])
    c2 = jnp.tanh(jnp.dot(hc, r_c2[slot], preferred_element_type=jnp.float32)
                  + cb2_ref[0])
    val = jnp.sum(c2 * vW_ref[0], axis=1) + vb_ref[e]
    act_ref[...] = act
    val_ref[...] = val


def _tc_experts(sched, vb_flat, x_pad, aW1, ab1, aW2, ab2,
                cW1, cb1, cW2, cb2, vW):
    nsp = len(sched) + 1
    sp = ["_"] * nsp

    def bias_spec():
        return pl.BlockSpec((1, 1, H), lambda i, *sp: (sp[0][i], 0, 0))

    grid_spec = pltpu.PrefetchScalarGridSpec(
        num_scalar_prefetch=nsp,
        grid=(NB,),
        in_specs=[
            pl.BlockSpec((BLK, D), lambda i, *sp: (i, 0)),
            pl.BlockSpec(memory_space=pl.ANY),
            bias_spec(),
            pl.BlockSpec(memory_space=pl.ANY),
            bias_spec(),
            pl.BlockSpec(memory_space=pl.ANY),
            bias_spec(),
            pl.BlockSpec(memory_space=pl.ANY),
            bias_spec(),
            bias_spec(),
        ],
        out_specs=[
            pl.BlockSpec((BLK, H), lambda i, *sp: (i, 0)),
            pl.BlockSpec((BLK,), lambda i, *sp: (i,)),
        ],
        scratch_shapes=[
            pltpu.VMEM((2, D, H), jnp.float32),
            pltpu.VMEM((2, H, H), jnp.float32),
            pltpu.VMEM((2, D, H), jnp.float32),
            pltpu.VMEM((2, H, H), jnp.float32),
            pltpu.SemaphoreType.DMA((2,)),
        ],
    )
    return pl.pallas_call(
        _tc_body,
        grid_spec=grid_spec,
        out_shape=[
            jax.ShapeDtypeStruct((NPAD, H), jnp.float32),
            jax.ShapeDtypeStruct((NPAD,), jnp.float32),
        ],
        compiler_params=pltpu.CompilerParams(
            dimension_semantics=("arbitrary",),
        ),
    )(*sched, vb_flat,
      x_pad,
      aW1, ab1.reshape(E, 1, H),
      aW2, ab2.reshape(E, 1, H),
      cW1, cb1.reshape(E, 1, H),
      cW2, cb2.reshape(E, 1, H),
      vW.reshape(E, 1, H))


def kernel(controller_ids, inputs, rnn_hxs, masks, aW1, ab1, aW2, ab2,
           cW1, cb1, cW2, cb2, vW, vb):
    ids = controller_ids.astype(jnp.int32)
    pos, sched = _routing(ids)

    sc_dispatch, sc_combine = _sc_kernels()
    pos2d = pos.reshape(N // GCH, GCH)
    x_pad = sc_dispatch(pos2d, inputs)
    act_pad, val_pad = _tc_experts(
        sched, vb.reshape(E), x_pad,
        aW1, ab1, aW2, ab2, cW1, cb1, cW2, cb2, vW)
    actor, value = sc_combine(pos2d, act_pad, val_pad)

    return (value.reshape(N, 1), actor, rnn_hxs)
